# baseline (speedup 1.0000x reference)
import jax
import jax.numpy as jnp
from jax.experimental import pallas as pl
from jax.experimental.pallas import tpu as pltpu


def _pos_body(col_hbm, row_hbm, out_ref, tabs, scratch, sem):
    nf = col_hbm.shape[1]
    h, w = scratch.shape[1], scratch.shape[2]
    bs = out_ref.shape[0]
    rep = scratch.shape[0]
    cin = pltpu.make_async_copy(col_hbm.at[pl.ds(0, w)], tabs.at[0], sem)
    rin = pltpu.make_async_copy(row_hbm.at[pl.ds(0, h)], tabs.at[1], sem)
    cin.start()
    rin.start()
    cin.wait()
    rin.wait()
    ce = tabs[0]
    re = tabs[1]
    pos_col = jnp.broadcast_to(ce[None, :, :], (h, w, nf))
    pos_row = jnp.broadcast_to(re[:, None, :], (h, w, nf))
    for r in range(rep):
        scratch[r, :, :, :nf] = pos_col
        scratch[r, :, :, nf:] = pos_row
    copies = [
        pltpu.make_async_copy(scratch, out_ref.at[pl.ds(b, rep)], sem)
        for b in range(0, bs, rep)
    ]
    for c in copies:
        c.start()
    for c in copies:
        c.wait()


def kernel(mask, feature_map, row_embed, col_embed):
    h, w = mask.shape[-2], mask.shape[-1]
    bs = mask.shape[0]
    nf = row_embed.shape[1]
    rep = 2
    q = pl.pallas_call(
        _pos_body,
        in_specs=[
            pl.BlockSpec(memory_space=pl.ANY),
            pl.BlockSpec(memory_space=pl.ANY),
        ],
        out_specs=pl.BlockSpec(memory_space=pl.ANY),
        out_shape=jax.ShapeDtypeStruct((bs, h, w, 2 * nf), jnp.float32),
        scratch_shapes=[
            pltpu.VMEM((2, w, nf), jnp.float32),
            pltpu.VMEM((rep, h, w, 2 * nf), jnp.float32),
            pltpu.SemaphoreType.DMA,
        ],
    )(col_embed, row_embed)
    return jnp.transpose(q, (0, 3, 1, 2))
